# baseline (device time: 41612 ns/iter reference)
import jax
import jax.numpy as jnp
from jax import lax
from jax.experimental import pallas as pl
from jax.experimental.pallas import tpu as pltpu

N_DEV = 4
N_LAYERS = 3
C = 4
NSLOT = 4
N_PHASES = 2 * N_LAYERS + 1


def kernel(x, Win0, Wout0, Win1, Wout1, Win2, Wout2):
    m, d = x.shape
    hid = Win0.shape[1]
    M = N_DEV * m
    mh = m // C

    def body(x_ref, win0_ref, wout0_ref, win1_ref, wout1_ref, win2_ref,
             wout2_ref, out_ref, sbuf, ownbuf, ybuf, ybuf16, rbuf,
             win16, wout16, wfin, wfout, send_sems, recv_sems, wsems):
        wdmas = []
        for i, (src, dst) in enumerate([
                (win0_ref, wfin.at[0]), (wout0_ref, wfout.at[0]),
                (win1_ref, wfin.at[1]), (wout1_ref, wfout.at[1]),
                (win2_ref, wfin.at[2]), (wout2_ref, wfout.at[2])]):
            cp = pltpu.make_async_copy(src, dst, wsems.at[i])
            cp.start()
            wdmas.append(cp)

        j = lax.axis_index("i")
        right = (j + 1) % N_DEV
        left = (j + N_DEV - 1) % N_DEV
        diag = (j + 2) % N_DEV
        senders = [left, right, diag]

        barrier_sem = pltpu.get_barrier_semaphore()
        for nbr in (left, right, diag):
            pl.semaphore_signal(
                barrier_sem, inc=1,
                device_id=(nbr,), device_id_type=pl.DeviceIdType.MESH,
            )
        pl.semaphore_wait(barrier_sem, 3)

        def start_send(p, tgt, slot, c, src):
            rd = pltpu.make_async_remote_copy(
                src_ref=src,
                dst_ref=rbuf.at[p, slot, c],
                send_sem=send_sems.at[p, slot, c],
                recv_sem=recv_sems.at[p, slot, c],
                device_id=(tgt,),
                device_id_type=pl.DeviceIdType.MESH,
            )
            rd.start()
            return rd

        def wait_recv(p, s, c):
            rd = pltpu.make_async_remote_copy(
                src_ref=rbuf.at[p, s, c],
                dst_ref=rbuf.at[p, s, c],
                send_sem=send_sems.at[p, s, c],
                recv_sem=recv_sems.at[p, s, c],
                device_id=(right,),
                device_id_type=pl.DeviceIdType.MESH,
            )
            rd.wait_recv()

        def block_partial(xblk16, l):
            h = jnp.dot(xblk16, win16[l, :, :],
                        preferred_element_type=jnp.float32)
            h16 = jnp.maximum(h, 0.0).astype(jnp.bfloat16)
            return jnp.dot(h16, wout16[l, :, :],
                           preferred_element_type=jnp.float32)

        def ag_and_rs_compute(p_ag, p_rs, lyr, ag_sends, rs_sends):
            for c in range(C):
                wait_recv(p_ag, 1, c)
                ag_sends.append(
                    start_send(p_ag, left, 2, c, rbuf.at[p_ag, 1, c]))
            for c in range(C):
                wait_recv(p_ag, 0, c)
                pfl = block_partial(rbuf[p_ag, 0, c, :, :], lyr)
                wait_recv(p_rs, 3, c)
                sbuf[0, c, :, :] = (
                    pfl + rbuf[p_rs, 3, c, :, :].astype(jnp.float32)
                ).astype(jnp.bfloat16)
                rs_sends.append(start_send(p_rs, right, 0, c,
                                           sbuf.at[0, c]))
                pfr = block_partial(rbuf[p_ag, 1, c, :, :], lyr)
                sbuf[1, c, :, :] = pfr.astype(jnp.bfloat16)
                rs_sends.append(start_send(p_rs, left, 1, c,
                                           sbuf.at[1, c]))
            for c in range(C):
                wait_recv(p_ag, 2, c)
                ownbuf[c, :, :] = block_partial(rbuf[p_ag, 2, c, :, :], lyr)

        for c in range(C):
            ybuf16[c, :, :] = x_ref[pl.ds(c * mh, mh), :].astype(jnp.bfloat16)
        ag_sends = []
        for c in range(C):
            ag_sends.append(start_send(0, right, 0, c, ybuf16.at[c]))
            ag_sends.append(start_send(0, left, 1, c, ybuf16.at[c]))
        wdmas[0].wait()
        wdmas[1].wait()
        win16[0, :, :] = wfin[0, :, :].astype(jnp.bfloat16)
        wout16[0, :, :] = wfout[0, :, :].astype(jnp.bfloat16)
        rs_sends = []
        for c in range(C):
            sbuf[2, c, :, :] = block_partial(ybuf16[c, :, :],
                                             0).astype(jnp.bfloat16)
            rs_sends.append(start_send(1, right, 3, c, sbuf.at[2, c]))
        for i in range(2, 6):
            wdmas[i].wait()
        win16[1, :, :] = wfin[1, :, :].astype(jnp.bfloat16)
        wout16[1, :, :] = wfout[1, :, :].astype(jnp.bfloat16)
        win16[2, :, :] = wfin[2, :, :].astype(jnp.bfloat16)
        wout16[2, :, :] = wfout[2, :, :].astype(jnp.bfloat16)
        ag_and_rs_compute(0, 1, 0, ag_sends, rs_sends)

        for l in range(N_LAYERS):
            p_rs = 2 * l + 1
            p_ag = 2 * l + 2
            last = l + 1 == N_LAYERS
            for rd in ag_sends:
                rd.wait_send()
            for rd in rs_sends:
                rd.wait_send()
            ag_sends = []
            rs_sends = []
            for c in range(C):
                wait_recv(p_rs, 0, c)
                wait_recv(p_rs, 1, c)
                ybuf[c, :, :] = (ownbuf[c, :, :]
                                 + rbuf[p_rs, 0, c, :, :].astype(jnp.float32)
                                 + rbuf[p_rs, 1, c, :, :].astype(jnp.float32))
                ybuf16[c, :, :] = ybuf[c, :, :].astype(jnp.bfloat16)
                ag_sends.append(start_send(p_ag, right, 0, c, ybuf16.at[c]))
                ag_sends.append(start_send(p_ag, left, 1, c, ybuf16.at[c]))
                if last:
                    out_ref[pl.ds(((j + 2) % N_DEV) * m + c * mh, mh), :] = (
                        ybuf[c, :, :])
                else:
                    sbuf[2, c, :, :] = block_partial(ybuf16[c, :, :],
                                                     l + 1).astype(jnp.bfloat16)
                    rs_sends.append(start_send(p_rs + 2, right, 3, c,
                                               sbuf.at[2, c]))
            if not last:
                ag_and_rs_compute(p_ag, p_rs + 2, l + 1, ag_sends, rs_sends)
            else:
                for c in range(C):
                    wait_recv(p_ag, 1, c)
                    ag_sends.append(
                        start_send(p_ag, left, 2, c, rbuf.at[p_ag, 1, c]))
                    out_ref[pl.ds(((right + 2) % N_DEV) * m + c * mh, mh),
                            :] = rbuf[p_ag, 1, c, :, :].astype(jnp.float32)
                for c in range(C):
                    wait_recv(p_ag, 0, c)
                    out_ref[pl.ds(((left + 2) % N_DEV) * m + c * mh, mh),
                            :] = rbuf[p_ag, 0, c, :, :].astype(jnp.float32)
                for c in range(C):
                    wait_recv(p_ag, 2, c)
                    out_ref[pl.ds(j * m + c * mh, mh), :] = (
                        rbuf[p_ag, 2, c, :, :].astype(jnp.float32))
                for rd in ag_sends:
                    rd.wait_send()

    return pl.pallas_call(
        body,
        out_shape=jax.ShapeDtypeStruct((M, d), jnp.float32),
        in_specs=[pl.BlockSpec(memory_space=pltpu.VMEM)]
        + [pl.BlockSpec(memory_space=pl.ANY)] * 6,
        out_specs=pl.BlockSpec(memory_space=pltpu.VMEM),
        scratch_shapes=[
            pltpu.VMEM((3, C, mh, d), jnp.bfloat16),
            pltpu.VMEM((C, mh, d), jnp.float32),
            pltpu.VMEM((C, mh, d), jnp.float32),
            pltpu.VMEM((C, mh, d), jnp.bfloat16),
            pltpu.VMEM((N_PHASES, NSLOT, C, mh, d), jnp.bfloat16),
            pltpu.VMEM((N_LAYERS, d, hid), jnp.bfloat16),
            pltpu.VMEM((N_LAYERS, hid, d), jnp.bfloat16),
            pltpu.VMEM((N_LAYERS, d, hid), jnp.float32),
            pltpu.VMEM((N_LAYERS, hid, d), jnp.float32),
            pltpu.SemaphoreType.DMA((N_PHASES, NSLOT, C)),
            pltpu.SemaphoreType.DMA((N_PHASES, NSLOT, C)),
            pltpu.SemaphoreType.DMA((6,)),
        ],
        compiler_params=pltpu.CompilerParams(collective_id=0),
    )(x, Win0, Wout0, Win1, Wout1, Win2, Wout2)


# device time: 38186 ns/iter; 1.0897x vs baseline; 1.0897x over previous
import jax
import jax.numpy as jnp
from jax import lax
from jax.experimental import pallas as pl
from jax.experimental.pallas import tpu as pltpu

N_DEV = 4
N_LAYERS = 3
C = 2
NSLOT = 4
N_PHASES = 2 * N_LAYERS + 1


def kernel(x, Win0, Wout0, Win1, Wout1, Win2, Wout2):
    m, d = x.shape
    hid = Win0.shape[1]
    M = N_DEV * m
    mh = m // C

    def body(x_ref, win0_ref, wout0_ref, win1_ref, wout1_ref, win2_ref,
             wout2_ref, out_ref, sbuf, ownbuf, ybuf16, rbuf,
             win16, wout16, wfin, wfout, send_sems, recv_sems, wsems):
        wdmas = []
        for i, (src, dst) in enumerate([
                (win0_ref, wfin.at[0]), (wout0_ref, wfout.at[0]),
                (win1_ref, wfin.at[1]), (wout1_ref, wfout.at[1]),
                (win2_ref, wfin.at[2]), (wout2_ref, wfout.at[2])]):
            cp = pltpu.make_async_copy(src, dst, wsems.at[i])
            cp.start()
            wdmas.append(cp)

        j = lax.axis_index("i")
        right = (j + 1) % N_DEV
        left = (j + N_DEV - 1) % N_DEV
        diag = (j + 2) % N_DEV
        senders = [left, right, diag]

        barrier_sem = pltpu.get_barrier_semaphore()
        for nbr in (left, right):
            pl.semaphore_signal(
                barrier_sem, inc=1,
                device_id=(nbr,), device_id_type=pl.DeviceIdType.MESH,
            )
        pl.semaphore_wait(barrier_sem, 2)

        def start_send(p, tgt, slot, c, src):
            rd = pltpu.make_async_remote_copy(
                src_ref=src,
                dst_ref=rbuf.at[p, slot, c],
                send_sem=send_sems.at[p, slot, c],
                recv_sem=recv_sems.at[p, slot, c],
                device_id=(tgt,),
                device_id_type=pl.DeviceIdType.MESH,
            )
            rd.start()
            return rd

        def wait_recv(p, s, c):
            rd = pltpu.make_async_remote_copy(
                src_ref=rbuf.at[p, s, c],
                dst_ref=rbuf.at[p, s, c],
                send_sem=send_sems.at[p, s, c],
                recv_sem=recv_sems.at[p, s, c],
                device_id=(right,),
                device_id_type=pl.DeviceIdType.MESH,
            )
            rd.wait_recv()

        def block_partial(xblk16, l):
            h = jnp.dot(xblk16, win16[l, :, :],
                        preferred_element_type=jnp.float32)
            h16 = jnp.maximum(h, 0.0).astype(jnp.bfloat16)
            return jnp.dot(h16, wout16[l, :, :],
                           preferred_element_type=jnp.float32)

        def ag_and_rs_compute(p_ag, p_rs, lyr, ag_sends, rs_sends):
            for c in range(C):
                wait_recv(p_ag, 1, c)
                ag_sends.append(
                    start_send(p_ag, left, 2, c, rbuf.at[p_ag, 1, c]))
            for c in range(C):
                wait_recv(p_ag, 0, c)
                pfl = block_partial(rbuf[p_ag, 0, c, :, :], lyr)
                wait_recv(p_rs, 3, c)
                sbuf[0, c, :, :] = (
                    pfl + rbuf[p_rs, 3, c, :, :].astype(jnp.float32)
                ).astype(jnp.bfloat16)
                rs_sends.append(start_send(p_rs, right, 0, c,
                                           sbuf.at[0, c]))
                pfr = block_partial(rbuf[p_ag, 1, c, :, :], lyr)
                sbuf[1, c, :, :] = pfr.astype(jnp.bfloat16)
                rs_sends.append(start_send(p_rs, left, 1, c,
                                           sbuf.at[1, c]))
            for c in range(C):
                wait_recv(p_ag, 2, c)
                ownbuf[c, :, :] = block_partial(rbuf[p_ag, 2, c, :, :], lyr)

        for c in range(C):
            ybuf16[c, :, :] = x_ref[pl.ds(c * mh, mh), :].astype(jnp.bfloat16)
        ag_sends = []
        for c in range(C):
            ag_sends.append(start_send(0, right, 0, c, ybuf16.at[c]))
            ag_sends.append(start_send(0, left, 1, c, ybuf16.at[c]))
        wdmas[0].wait()
        wdmas[1].wait()
        win16[0, :, :] = wfin[0, :, :].astype(jnp.bfloat16)
        wout16[0, :, :] = wfout[0, :, :].astype(jnp.bfloat16)
        rs_sends = []
        for c in range(C):
            sbuf[2, c, :, :] = block_partial(ybuf16[c, :, :],
                                             0).astype(jnp.bfloat16)
            rs_sends.append(start_send(1, right, 3, c, sbuf.at[2, c]))
        for i in range(2, 6):
            wdmas[i].wait()
        win16[1, :, :] = wfin[1, :, :].astype(jnp.bfloat16)
        wout16[1, :, :] = wfout[1, :, :].astype(jnp.bfloat16)
        win16[2, :, :] = wfin[2, :, :].astype(jnp.bfloat16)
        wout16[2, :, :] = wfout[2, :, :].astype(jnp.bfloat16)
        ag_and_rs_compute(0, 1, 0, ag_sends, rs_sends)

        for l in range(N_LAYERS):
            p_rs = 2 * l + 1
            p_ag = 2 * l + 2
            last = l + 1 == N_LAYERS
            for rd in ag_sends:
                rd.wait_send()
            for rd in rs_sends:
                rd.wait_send()
            ag_sends = []
            rs_sends = []
            for c in range(C):
                wait_recv(p_rs, 0, c)
                wait_recv(p_rs, 1, c)
                ysum = (ownbuf[c, :, :]
                        + rbuf[p_rs, 0, c, :, :].astype(jnp.float32)
                        + rbuf[p_rs, 1, c, :, :].astype(jnp.float32))
                ybuf16[c, :, :] = ysum.astype(jnp.bfloat16)
                ag_sends.append(start_send(p_ag, right, 0, c, ybuf16.at[c]))
                ag_sends.append(start_send(p_ag, left, 1, c, ybuf16.at[c]))
                if last:
                    out_ref[pl.ds(((j + 2) % N_DEV) * m + c * mh, mh), :] = (
                        ysum)
                else:
                    sbuf[2, c, :, :] = block_partial(ybuf16[c, :, :],
                                                     l + 1).astype(jnp.bfloat16)
                    rs_sends.append(start_send(p_rs + 2, right, 3, c,
                                               sbuf.at[2, c]))
            if not last:
                ag_and_rs_compute(p_ag, p_rs + 2, l + 1, ag_sends, rs_sends)
            else:
                for c in range(C):
                    wait_recv(p_ag, 1, c)
                    ag_sends.append(
                        start_send(p_ag, left, 2, c, rbuf.at[p_ag, 1, c]))
                    out_ref[pl.ds(((right + 2) % N_DEV) * m + c * mh, mh),
                            :] = rbuf[p_ag, 1, c, :, :].astype(jnp.float32)
                for c in range(C):
                    wait_recv(p_ag, 0, c)
                    out_ref[pl.ds(((left + 2) % N_DEV) * m + c * mh, mh),
                            :] = rbuf[p_ag, 0, c, :, :].astype(jnp.float32)
                for c in range(C):
                    wait_recv(p_ag, 2, c)
                    out_ref[pl.ds(j * m + c * mh, mh), :] = (
                        rbuf[p_ag, 2, c, :, :].astype(jnp.float32))
                for rd in ag_sends:
                    rd.wait_send()

    return pl.pallas_call(
        body,
        out_shape=jax.ShapeDtypeStruct((M, d), jnp.float32),
        in_specs=[pl.BlockSpec(memory_space=pltpu.VMEM)]
        + [pl.BlockSpec(memory_space=pl.ANY)] * 6,
        out_specs=pl.BlockSpec(memory_space=pltpu.VMEM),
        scratch_shapes=[
            pltpu.VMEM((3, C, mh, d), jnp.bfloat16),
            pltpu.VMEM((C, mh, d), jnp.float32),
            pltpu.VMEM((C, mh, d), jnp.bfloat16),
            pltpu.VMEM((N_PHASES, NSLOT, C, mh, d), jnp.bfloat16),
            pltpu.VMEM((N_LAYERS, d, hid), jnp.bfloat16),
            pltpu.VMEM((N_LAYERS, hid, d), jnp.bfloat16),
            pltpu.VMEM((N_LAYERS, d, hid), jnp.float32),
            pltpu.VMEM((N_LAYERS, hid, d), jnp.float32),
            pltpu.SemaphoreType.DMA((N_PHASES, NSLOT, C)),
            pltpu.SemaphoreType.DMA((N_PHASES, NSLOT, C)),
            pltpu.SemaphoreType.DMA((6,)),
        ],
        compiler_params=pltpu.CompilerParams(collective_id=0),
    )(x, Win0, Wout0, Win1, Wout1, Win2, Wout2)
